# baseline (speedup 1.0000x reference)
import numpy as np
import jax
import jax.numpy as jnp
from jax.experimental import pallas as pl
from jax.experimental.pallas import tpu as pltpu

V = 100000
ROWS = 512
P = 8
STEPS = ROWS // P

_KU = (np.uint32(1832780943), np.uint32(270669613))
_KS = (np.uint32(64467757), np.uint32(2916123636))

_C999 = np.float32(0.999 - 1e-12)
_EPS = np.float32(1e-12)
_TINY = np.float32(np.finfo(np.float32).tiny)
_ONE_MINUS_TINY = np.float32(1.0) - _TINY
_LOG_EPS = np.float32(1e-30)

_ROT_A = (13, 15, 26, 6)
_ROT_B = (17, 29, 16, 24)


def _threefry_bits(k1, k2, lo):
    k3 = np.uint32(np.uint32(k1) ^ np.uint32(k2) ^ np.uint32(0x1BD11BDA))
    ks = [k1, k2, k3]
    x0 = jnp.full_like(lo, k1)
    x1 = lo + k2
    rots = [_ROT_A, _ROT_B]
    for i in range(5):
        for r in rots[0]:
            x0 = x0 + x1
            x1 = (x1 << np.uint32(r)) | (x1 >> np.uint32(32 - r))
            x1 = x1 ^ x0
        x0 = x0 + ks[1]
        x1 = x1 + ks[2] + np.uint32(i + 1)
        ks = ks[1:] + ks[:1]
        rots = rots[1:] + rots[:1]
    return x0 ^ x1


def _bits_to_u01(bits):
    fb = (bits >> np.uint32(9)) | np.uint32(0x3F800000)
    return jax.lax.bitcast_convert_type(fb, jnp.float32) - np.float32(1.0)


def _stgs_body(x_ref, y1_ref, y2_ref, ids_ref):
    i = pl.program_id(0)
    row = jax.lax.broadcasted_iota(jnp.uint32, (P, V), 0)
    col = jax.lax.broadcasted_iota(jnp.uint32, (P, V), 1)
    idx = (jnp.uint32(i * P) + row) * jnp.uint32(V) + col

    u = _bits_to_u01(_threefry_bits(_KU[0], _KU[1], idx)) * _C999 + _EPS
    gl = x_ref[...] + (-jnp.log(-jnp.log(u)))

    m = jnp.max(gl, axis=1, keepdims=True)
    e = jnp.exp(gl - m)
    s = jnp.sum(e, axis=1, keepdims=True)
    y = e / s
    y1_ref[...] = y
    y2_ref[...] = y

    u2 = _bits_to_u01(_threefry_bits(_KS[0], _KS[1], idx))
    u2 = jnp.maximum(_TINY, u2 * _ONE_MINUS_TINY + _TINY)
    val = jnp.log(y + _LOG_EPS) + (-jnp.log(-jnp.log(u2)))
    vmax = jnp.max(val, axis=1, keepdims=True)
    coli = jax.lax.broadcasted_iota(jnp.int32, (P, V), 1)
    cand = jnp.where(val == vmax, coli, jnp.int32(V))
    ids = jnp.min(cand, axis=1, keepdims=True).astype(jnp.float32)
    ids_ref[...] = ids.reshape(1, P, 1)


def kernel(x):
    xf = x.reshape(ROWS, V)
    y1, y2, ids = pl.pallas_call(
        _stgs_body,
        grid=(STEPS,),
        in_specs=[pl.BlockSpec((P, V), lambda i: (i, 0))],
        out_specs=[
            pl.BlockSpec((P, V), lambda i: (i, 0)),
            pl.BlockSpec((P, V), lambda i: (i, 0)),
            pl.BlockSpec((1, P, 1), lambda i: (i, 0, 0)),
        ],
        out_shape=[
            jax.ShapeDtypeStruct((ROWS, V), jnp.float32),
            jax.ShapeDtypeStruct((ROWS, V), jnp.float32),
            jax.ShapeDtypeStruct((STEPS, P, 1), jnp.float32),
        ],
        compiler_params=pltpu.CompilerParams(
            dimension_semantics=("arbitrary",),
        ),
    )(xf)
    diff_ids = ids.reshape(64, 8)
    y_soft = y1.reshape(64, 8, V)
    one_hot = y2.reshape(64, 8, V)
    eff_temperature = jnp.ones((1,), jnp.float32)
    return (diff_ids, one_hot, eff_temperature, y_soft)
